# baseline (speedup 1.0000x reference)
import functools

import jax
import jax.numpy as jnp
from jax import lax
from jax.experimental import pallas as pl
from jax.experimental.pallas import tpu as pltpu
from jax.experimental.pallas import tpu_sc as plsc

DIM = 128
BATCH = 16384
LANES = 16
CHUNK = 64
SC_ROWS = 8192
TC_BLOCK = 2048


def _hour_from_unix(tv):
    n = lax.shift_right_logical(tv, 7)
    q = (n.astype(jnp.float32) * jnp.float32(1.0 / 675.0)).astype(jnp.int32)
    r = tv - q * 86400
    r = jnp.where(r < 0, r + 86400, r)
    r = jnp.where(r >= 86400, r - 86400, r)
    h = (r.astype(jnp.float32) * jnp.float32(1.0 / 3600.0)).astype(jnp.int32)
    rem = r - h * 3600
    h = jnp.where(rem < 0, h - 1, h)
    rem = jnp.where(rem < 0, rem + 3600, rem)
    h = jnp.where(rem >= 3600, h + 1, h)
    return h


def _sc_gather(t_sc, table):
    info = plsc.get_sparse_core_info()
    nc, ns = info.num_cores, info.num_subcores
    nw = nc * ns
    bpw = SC_ROWS // nw
    nchunks = bpw // CHUNK

    mesh = plsc.VectorSubcoreMesh(core_axis_name="c", subcore_axis_name="s")

    @functools.partial(
        pl.kernel,
        mesh=mesh,
        out_type=jax.ShapeDtypeStruct((SC_ROWS, DIM), jnp.float32),
        scratch_types=[
            pltpu.VMEM((bpw,), jnp.int32),
            pltpu.VMEM((nchunks, CHUNK), jnp.int32),
            pltpu.VMEM((nchunks, CHUNK, DIM), jnp.float32),
            pltpu.VMEM_SHARED((24, DIM), jnp.float32),
        ] + [pltpu.SemaphoreType.DMA] * (nchunks + 1),
    )
    def sc_lookup(t_hbm, tab_hbm, out_hbm, t_v, idx_v, rows_v, tab_sh, *sems):
        gsems, ssem = sems[:nchunks], sems[nchunks]
        sid = lax.axis_index("s")
        wid = sid * nc + lax.axis_index("c")
        base = wid * bpw

        @pl.when(sid == 0)
        def _stage_table():
            pltpu.sync_copy(tab_hbm, tab_sh)

        pltpu.sync_copy(t_hbm.at[pl.ds(base, bpw)], t_v)
        plsc.subcore_barrier()

        gathers = []
        for j in range(nchunks):
            def _compute(k, carry, j=j):
                tv = t_v[pl.ds(j * CHUNK + k * LANES, LANES)]
                idx_v[j, pl.ds(k * LANES, LANES)] = _hour_from_unix(tv)
                return carry

            lax.fori_loop(0, CHUNK // LANES, _compute, 0)
            gathers.append(
                pltpu.async_copy(tab_sh.at[idx_v.at[j]], rows_v.at[j], gsems[j]))

        scatters = []
        for j in range(nchunks):
            gathers[j].wait()
            scatters.append(
                pltpu.async_copy(rows_v.at[j],
                                 out_hbm.at[pl.ds(base + j * CHUNK, CHUNK)], ssem))
        for j in range(nchunks):
            scatters[j].wait()

    return sc_lookup(t_sc, table)


def _tc_hours_kernel(t_ref, h_ref):
    h_ref[...] = _hour_from_unix(t_ref[...])


def _tc_assemble_kernel(h_ref, tab_ref, sc_ref, out_ref):
    i = pl.program_id(0)
    n_sc = SC_ROWS // TC_BLOCK

    @pl.when(i < n_sc)
    def _copy():
        out_ref[...] = sc_ref[...]

    @pl.when(i >= n_sc)
    def _matmul():
        h = h_ref[...]
        hour_ids = lax.broadcasted_iota(jnp.int32, (1, 24), 1)
        onehot = (h == hour_ids).astype(jnp.float32)
        out_ref[...] = jnp.dot(onehot, tab_ref[...],
                               preferred_element_type=jnp.float32)


def kernel(t, week_emb, day_emb, month_emb, hour_emb):
    del week_emb, day_emb, month_emb

    sc_out = _sc_gather(t[:SC_ROWS], hour_emb)

    h_dense = pl.pallas_call(
        _tc_hours_kernel,
        out_shape=jax.ShapeDtypeStruct((BATCH // DIM, DIM), jnp.int32),
    )(t.reshape(BATCH // DIM, DIM))
    h_col = h_dense.reshape(BATCH, 1)

    n_blocks = BATCH // TC_BLOCK
    n_sc = SC_ROWS // TC_BLOCK
    out = pl.pallas_call(
        _tc_assemble_kernel,
        grid=(n_blocks,),
        in_specs=[
            pl.BlockSpec((TC_BLOCK, 1), lambda i: (i, 0)),
            pl.BlockSpec((24, DIM), lambda i: (0, 0)),
            pl.BlockSpec((TC_BLOCK, DIM),
                         lambda i: (jnp.minimum(i, n_sc - 1), 0)),
        ],
        out_specs=pl.BlockSpec((TC_BLOCK, DIM), lambda i: (i, 0)),
        out_shape=jax.ShapeDtypeStruct((BATCH, DIM), jnp.float32),
    )(h_col, hour_emb, sc_out)
    return out


# speedup vs baseline: 1.3733x; 1.3733x over previous
import functools

import jax
import jax.numpy as jnp
from jax import lax
from jax.experimental import pallas as pl
from jax.experimental.pallas import tpu as pltpu
from jax.experimental.pallas import tpu_sc as plsc

DIM = 128
BATCH = 16384
LANES = 16
CHUNK = 32


def _hour_from_unix(tv):
    n = lax.shift_right_logical(tv, 7)
    q = (n.astype(jnp.float32) * jnp.float32(1.0 / 675.0)).astype(jnp.int32)
    r = tv - q * 86400
    r = jnp.where(r < 0, r + 86400, r)
    r = jnp.where(r >= 86400, r - 86400, r)
    h = (r.astype(jnp.float32) * jnp.float32(1.0 / 3600.0)).astype(jnp.int32)
    rem = r - h * 3600
    h = jnp.where(rem < 0, h - 1, h)
    rem = jnp.where(rem < 0, rem + 3600, rem)
    h = jnp.where(rem >= 3600, h + 1, h)
    return h


def kernel(t, week_emb, day_emb, month_emb, hour_emb):
    del week_emb, day_emb, month_emb
    info = plsc.get_sparse_core_info()
    nc, ns = info.num_cores, info.num_subcores
    nw = nc * ns
    bpw = BATCH // nw
    nchunks = bpw // CHUNK

    mesh = plsc.VectorSubcoreMesh(core_axis_name="c", subcore_axis_name="s")

    @functools.partial(
        pl.kernel,
        mesh=mesh,
        out_type=jax.ShapeDtypeStruct((BATCH, DIM), jnp.float32),
        scratch_types=[
            pltpu.VMEM((bpw,), jnp.int32),
            pltpu.VMEM((nchunks, CHUNK), jnp.int32),
            pltpu.VMEM((nchunks, CHUNK, DIM), jnp.float32),
            pltpu.VMEM_SHARED((24, DIM), jnp.float32),
        ] + [pltpu.SemaphoreType.DMA] * (nchunks + 1),
    )
    def sc_lookup(t_hbm, tab_hbm, out_hbm, t_v, idx_v, rows_v, tab_sh, *sems):
        gsems, ssem = sems[:nchunks], sems[nchunks]
        sid = lax.axis_index("s")
        wid = sid * nc + lax.axis_index("c")
        base = wid * bpw

        @pl.when(sid == 0)
        def _stage_table():
            pltpu.sync_copy(tab_hbm, tab_sh)

        pltpu.sync_copy(t_hbm.at[pl.ds(base, bpw)], t_v)
        plsc.subcore_barrier()

        gathers = []
        for j in range(nchunks):
            def _compute(k, carry, j=j):
                tv = t_v[pl.ds(j * CHUNK + k * LANES, LANES)]
                idx_v[j, pl.ds(k * LANES, LANES)] = _hour_from_unix(tv)
                return carry

            lax.fori_loop(0, CHUNK // LANES, _compute, 0)
            gathers.append(
                pltpu.async_copy(tab_sh.at[idx_v.at[j]], rows_v.at[j], gsems[j]))

        scatters = []
        for j in range(nchunks):
            gathers[j].wait()
            scatters.append(
                pltpu.async_copy(rows_v.at[j],
                                 out_hbm.at[pl.ds(base + j * CHUNK, CHUNK)], ssem))
        for j in range(nchunks):
            scatters[j].wait()

    return sc_lookup(t, hour_emb)


# speedup vs baseline: 1.4149x; 1.0302x over previous
import functools

import jax
import jax.numpy as jnp
from jax import lax
from jax.experimental import pallas as pl
from jax.experimental.pallas import tpu as pltpu
from jax.experimental.pallas import tpu_sc as plsc

DIM = 128
BATCH = 16384
LANES = 16
CHUNK = 64


def _hour_from_unix(tv):
    n = lax.shift_right_logical(tv, 7)
    q = (n.astype(jnp.float32) * jnp.float32(1.0 / 675.0)).astype(jnp.int32)
    r = tv - q * 86400
    r = jnp.where(r < 0, r + 86400, r)
    r = jnp.where(r >= 86400, r - 86400, r)
    h = (r.astype(jnp.float32) * jnp.float32(1.0 / 3600.0)).astype(jnp.int32)
    rem = r - h * 3600
    h = jnp.where(rem < 0, h - 1, h)
    rem = jnp.where(rem < 0, rem + 3600, rem)
    h = jnp.where(rem >= 3600, h + 1, h)
    return h


def kernel(t, week_emb, day_emb, month_emb, hour_emb):
    del week_emb, day_emb, month_emb
    info = plsc.get_sparse_core_info()
    nc, ns = info.num_cores, info.num_subcores
    nw = nc * ns
    bpw = BATCH // nw
    nchunks = bpw // CHUNK

    mesh = plsc.VectorSubcoreMesh(core_axis_name="c", subcore_axis_name="s")

    @functools.partial(
        pl.kernel,
        mesh=mesh,
        out_type=jax.ShapeDtypeStruct((BATCH, DIM), jnp.float32),
        scratch_types=[
            pltpu.VMEM((bpw,), jnp.int32),
            pltpu.VMEM((nchunks, CHUNK), jnp.int32),
            pltpu.VMEM((nchunks, CHUNK, DIM), jnp.float32),
            pltpu.VMEM_SHARED((24, DIM), jnp.float32),
        ] + [pltpu.SemaphoreType.DMA] * (nchunks + 2),
    )
    def sc_lookup(t_hbm, tab_hbm, out_hbm, t_v, idx_v, rows_v, tab_sh, *sems):
        gsems, ssem, stsem = sems[:nchunks], sems[nchunks], sems[nchunks + 1]
        sid = lax.axis_index("s")
        wid = sid * nc + lax.axis_index("c")
        base = wid * bpw

        @pl.when(sid == 0)
        def _stage_table():
            pltpu.async_copy(tab_hbm, tab_sh, stsem)

        pltpu.sync_copy(t_hbm.at[pl.ds(base, bpw)], t_v)

        @pl.when(sid == 0)
        def _stage_wait():
            pltpu.make_async_copy(tab_hbm, tab_sh, stsem).wait()

        gathers = []
        for j in range(nchunks):
            def _compute(k, carry, j=j):
                tv = t_v[pl.ds(j * CHUNK + k * LANES, LANES)]
                idx_v[j, pl.ds(k * LANES, LANES)] = _hour_from_unix(tv)
                return carry

            lax.fori_loop(0, CHUNK // LANES, _compute, 0)
            if j == 0:
                plsc.subcore_barrier()
            gathers.append(
                pltpu.async_copy(tab_sh.at[idx_v.at[j]], rows_v.at[j], gsems[j]))

        scatters = []
        for j in range(nchunks):
            gathers[j].wait()
            scatters.append(
                pltpu.async_copy(rows_v.at[j],
                                 out_hbm.at[pl.ds(base + j * CHUNK, CHUNK)], ssem))
        for j in range(nchunks):
            scatters[j].wait()

    return sc_lookup(t, hour_emb)
